# Initial kernel scaffold; baseline (speedup 1.0000x reference)
import jax
import jax.numpy as jnp
from jax.experimental import pallas as pl


def kernel(x, edge_index, batch, solution_feature, emb, W1l, W1r, b1, W2l, W2r, b2, Wm1, bm1, Wm2, bm2, Wm3, bm3):
    raise NotImplementedError("write your pallas kernel here")


# speedup vs baseline: 4.6709x; 4.6709x over previous
import jax
import jax.numpy as jnp
from jax import lax
from jax.experimental import pallas as pl
from jax.experimental.pallas import tpu as pltpu
from jax.experimental.pallas import tpu_sc as plsc

_NC = 2
_NS = 16
_CHUNK = 80
_CW = 128
_CR = 160


def _node_pad(n_nodes):
    return -(-n_nodes // (_NS * _CR)) * (_NS * _CR)


def _make_edge_agg(n_nodes, n_edges, feat):
    nw = _NC * _NS
    epw = n_edges // nw
    nchunks = epw // _CHUNK
    npad = _node_pad(n_nodes)
    rps = npad // _NS
    niw = rps // _CR

    mesh = plsc.VectorSubcoreMesh(core_axis_name="c", subcore_axis_name="s")

    def body(h_hbm, src_hbm, dst_hbm, zf_hbm, agg_hbm,
             sidx, didx, rows, stage, agg_sh, sem):
        cid = lax.axis_index("c")
        sid = lax.axis_index("s")
        wid = sid * _NC + cid
        rb = sid * rps

        pltpu.sync_copy(zf_hbm, stage)
        for j in range(niw):
            pltpu.sync_copy(stage, agg_sh.at[pl.ds(rb + j * _CR, _CR)])
        plsc.subcore_barrier()

        ebase = wid * epw

        @pl.loop(0, nchunks)
        def step(i):
            eb = ebase + i * _CHUNK
            pltpu.sync_copy(src_hbm.at[pl.ds(eb, _CHUNK)], sidx)
            pltpu.sync_copy(dst_hbm.at[pl.ds(eb, _CHUNK)], didx)
            pltpu.async_copy(h_hbm.at[sidx], rows, sem).wait()
            pltpu.sync_copy(rows, agg_sh.at[didx], add=True)

        plsc.subcore_barrier()
        for j in range(niw):
            ro = rb + j * _CR
            pltpu.sync_copy(agg_sh.at[pl.ds(ro, _CR)], stage)
            pltpu.sync_copy(stage, agg_hbm.at[pl.ds(cid * npad + ro, _CR)])

    return pl.kernel(
        body,
        out_type=jax.ShapeDtypeStruct((_NC * npad, feat), jnp.float32),
        mesh=mesh,
        scratch_types=[
            pltpu.VMEM((_CHUNK,), jnp.int32),
            pltpu.VMEM((_CHUNK,), jnp.int32),
            pltpu.VMEM((_CHUNK, feat), jnp.float32),
            pltpu.VMEM((_CR, feat), jnp.float32),
            pltpu.VMEM_SHARED((npad, feat), jnp.float32),
            pltpu.SemaphoreType.DMA,
        ])


def _make_edge_cnt(n_nodes, n_edges):
    nw = _NC * _NS
    epw = n_edges // nw
    nchunks = epw // _CHUNK
    npad = _node_pad(n_nodes)
    rps = npad // _NS
    niw = rps // _CR

    mesh = plsc.VectorSubcoreMesh(core_axis_name="c", subcore_axis_name="s")

    def body(dst_hbm, zc_hbm, ones_hbm, cnt_hbm,
             didx, ones_v, cstage, cnt_sh):
        cid = lax.axis_index("c")
        sid = lax.axis_index("s")
        wid = sid * _NC + cid
        rb = sid * rps

        pltpu.sync_copy(zc_hbm, cstage)
        pltpu.sync_copy(ones_hbm, ones_v)
        for j in range(niw):
            pltpu.sync_copy(cstage, cnt_sh.at[pl.ds(rb + j * _CR, _CR)])
        plsc.subcore_barrier()

        ebase = wid * epw

        @pl.loop(0, nchunks)
        def step(i):
            eb = ebase + i * _CHUNK
            pltpu.sync_copy(dst_hbm.at[pl.ds(eb, _CHUNK)], didx)
            pltpu.sync_copy(ones_v, cnt_sh.at[didx], add=True)

        plsc.subcore_barrier()
        for j in range(niw):
            ro = rb + j * _CR
            pltpu.sync_copy(cnt_sh.at[pl.ds(ro, _CR)], cstage)
            pltpu.sync_copy(cstage, cnt_hbm.at[pl.ds(cid * npad + ro, _CR)])

    return pl.kernel(
        body,
        out_type=jax.ShapeDtypeStruct((_NC * npad, _CW), jnp.float32),
        mesh=mesh,
        scratch_types=[
            pltpu.VMEM((_CHUNK,), jnp.int32),
            pltpu.VMEM((_CHUNK, _CW), jnp.float32),
            pltpu.VMEM((_CR, _CW), jnp.float32),
            pltpu.VMEM_SHARED((npad, _CW), jnp.float32),
        ])


def _embed_tc(x2, emb):
    n = x2.shape[0]
    t, e = emb.shape
    blk = 1000

    def body(x_ref, emb_ref, o_ref):
        x = x_ref[...]
        oh = (x == lax.broadcasted_iota(jnp.int32, (blk, t), 1)).astype(
            jnp.float32)
        o_ref[...] = jnp.dot(oh, emb_ref[...],
                             preferred_element_type=jnp.float32,
                     precision=lax.Precision.HIGHEST)

    return pl.pallas_call(
        body,
        grid=(n // blk,),
        in_specs=[pl.BlockSpec((blk, 1), lambda i: (i, 0)),
                  pl.BlockSpec((t, e), lambda i: (0, 0))],
        out_specs=pl.BlockSpec((blk, e), lambda i: (i, 0)),
        out_shape=jax.ShapeDtypeStruct((n, e), jnp.float32),
    )(x2, emb)


def _conv_tc(agg, cnt, h, wl_t, wr_t, b2d):
    n, f = h.shape
    npad = agg.shape[0] // _NC
    blk = 1000

    def body(a_ref, c_ref, h_ref, wl_ref, wr_ref, b_ref, o_ref):
        asum = a_ref[0] + a_ref[1]
        csum = c_ref[0][:, :1] + c_ref[1][:, :1]
        mean = asum / jnp.maximum(csum, 1.0)
        o = jnp.dot(mean, wl_ref[...], preferred_element_type=jnp.float32,
                     precision=lax.Precision.HIGHEST)
        o += jnp.dot(h_ref[...], wr_ref[...],
                     preferred_element_type=jnp.float32,
                     precision=lax.Precision.HIGHEST)
        o_ref[...] = jnp.maximum(o + b_ref[...], 0.0)

    agg3 = agg.reshape(_NC, npad, f)
    cnt3 = cnt.reshape(_NC, npad, _CW)
    return pl.pallas_call(
        body,
        grid=(n // blk,),
        in_specs=[pl.BlockSpec((_NC, blk, f), lambda i: (0, i, 0)),
                  pl.BlockSpec((_NC, blk, _CW), lambda i: (0, i, 0)),
                  pl.BlockSpec((blk, f), lambda i: (i, 0)),
                  pl.BlockSpec((f, f), lambda i: (0, 0)),
                  pl.BlockSpec((f, f), lambda i: (0, 0)),
                  pl.BlockSpec((1, f), lambda i: (0, 0))],
        out_specs=pl.BlockSpec((blk, f), lambda i: (i, 0)),
        out_shape=jax.ShapeDtypeStruct((n, f), jnp.float32),
    )(agg3, cnt3, h, wl_t, wr_t, b2d)


def _conv_pool_tc(agg, cnt, h, wl_t, wr_t, b2d, batch2, num_graphs):
    n, f = h.shape
    npad = agg.shape[0] // _NC
    blk = 1000

    def body(a_ref, c_ref, h_ref, wl_ref, wr_ref, b_ref, bt_ref,
             gs_ref, gc_ref):
        i = pl.program_id(0)
        asum = a_ref[0] + a_ref[1]
        csum = c_ref[0][:, :1] + c_ref[1][:, :1]
        mean = asum / jnp.maximum(csum, 1.0)
        o = jnp.dot(mean, wl_ref[...], preferred_element_type=jnp.float32,
                     precision=lax.Precision.HIGHEST)
        o += jnp.dot(h_ref[...], wr_ref[...],
                     preferred_element_type=jnp.float32,
                     precision=lax.Precision.HIGHEST)
        h2 = jnp.maximum(o + b_ref[...], 0.0)
        bt = bt_ref[...]
        oh = (bt == lax.broadcasted_iota(jnp.int32, (blk, num_graphs),
                                         1)).astype(jnp.float32)
        ps = lax.dot_general(oh, h2, (((0,), (0,)), ((), ())),
                             preferred_element_type=jnp.float32,
                     precision=lax.Precision.HIGHEST)
        pc = jnp.broadcast_to(jnp.sum(oh, axis=0)[:, None], (num_graphs, f))

        @pl.when(i == 0)
        def _():
            gs_ref[...] = ps
            gc_ref[...] = pc

        @pl.when(i != 0)
        def _():
            gs_ref[...] += ps
            gc_ref[...] += pc

    agg3 = agg.reshape(_NC, npad, f)
    cnt3 = cnt.reshape(_NC, npad, _CW)
    return pl.pallas_call(
        body,
        grid=(n // blk,),
        in_specs=[pl.BlockSpec((_NC, blk, f), lambda i: (0, i, 0)),
                  pl.BlockSpec((_NC, blk, _CW), lambda i: (0, i, 0)),
                  pl.BlockSpec((blk, f), lambda i: (i, 0)),
                  pl.BlockSpec((f, f), lambda i: (0, 0)),
                  pl.BlockSpec((f, f), lambda i: (0, 0)),
                  pl.BlockSpec((1, f), lambda i: (0, 0)),
                  pl.BlockSpec((blk, 1), lambda i: (i, 0))],
        out_specs=[pl.BlockSpec((num_graphs, f), lambda i: (0, 0)),
                   pl.BlockSpec((num_graphs, f), lambda i: (0, 0))],
        out_shape=[jax.ShapeDtypeStruct((num_graphs, f), jnp.float32),
                   jax.ShapeDtypeStruct((num_graphs, f), jnp.float32)],
    )(agg3, cnt3, h, wl_t, wr_t, b2d, batch2)


def _mlp_tc(gs, gc, sol, w1g_t, w1s_t, b1_2d, w2_t, b2_2d, w3_t, b3_2d):
    ng, f = gs.shape
    o = w3_t.shape[1]

    def body(gs_ref, gc_ref, sol_ref, w1g, w1s, b1r, w2, b2r, w3, b3r,
             o_ref):
        g = gs_ref[...] / jnp.maximum(gc_ref[...], 1.0)
        z = jnp.dot(g, w1g[...], preferred_element_type=jnp.float32,
                     precision=lax.Precision.HIGHEST)
        z += jnp.dot(sol_ref[...], w1s[...],
                     preferred_element_type=jnp.float32,
                     precision=lax.Precision.HIGHEST)
        z = jnp.maximum(z + b1r[...], 0.0)
        z = jnp.maximum(
            jnp.dot(z, w2[...], preferred_element_type=jnp.float32,
                     precision=lax.Precision.HIGHEST)
            + b2r[...], 0.0)
        o_ref[...] = (jnp.dot(z, w3[...], preferred_element_type=jnp.float32,
                     precision=lax.Precision.HIGHEST)
                      + b3r[...])

    return pl.pallas_call(
        body,
        out_shape=jax.ShapeDtypeStruct((ng, o), jnp.float32),
    )(gs, gc, sol, w1g_t, w1s_t, b1_2d, w2_t, b2_2d, w3_t, b3_2d)


def kernel(x, edge_index, batch, solution_feature, emb, W1l, W1r, b1,
           W2l, W2r, b2, Wm1, bm1, Wm2, bm2, Wm3, bm3):
    n = x.shape[0]
    n_edges = edge_index.shape[1]
    feat = emb.shape[1]
    num_graphs = solution_feature.shape[0]

    x2 = x.astype(jnp.int32).reshape(n, 1)
    src = edge_index[0].astype(jnp.int32)
    dst = edge_index[1].astype(jnp.int32)
    batch2 = batch.astype(jnp.int32).reshape(n, 1)

    zf = jnp.zeros((_CR, feat), jnp.float32)
    zc = jnp.zeros((_CR, _CW), jnp.float32)
    ones = jnp.ones((_CHUNK, _CW), jnp.float32)

    edge_agg = _make_edge_agg(n, n_edges, feat)
    cnt = _make_edge_cnt(n, n_edges)(dst, zc, ones)

    h0 = _embed_tc(x2, emb)
    agg1 = edge_agg(h0, src, dst, zf)
    h1 = _conv_tc(agg1, cnt, h0, W1l.T, W1r.T, b1.reshape(1, -1))
    agg2 = edge_agg(h1, src, dst, zf)
    gs, gc = _conv_pool_tc(agg2, cnt, h1, W2l.T, W2r.T, b2.reshape(1, -1),
                           batch2, num_graphs)
    out = _mlp_tc(gs, gc, solution_feature,
                  Wm1.T[:feat], Wm1.T[feat:], bm1.reshape(1, -1),
                  Wm2.T, bm2.reshape(1, -1),
                  Wm3.T, bm3.reshape(1, -1))
    return out


# speedup vs baseline: 7.8851x; 1.6881x over previous
import jax
import jax.numpy as jnp
from jax import lax
from jax.experimental import pallas as pl
from jax.experimental.pallas import tpu as pltpu
from jax.experimental.pallas import tpu_sc as plsc

_NC = 2
_NS = 16
_CHUNK = 40
_CW = 128
_CR = 40


def _node_pad(n_nodes):
    return -(-n_nodes // (_NS * _CR)) * (_NS * _CR)


def _make_edge_agg(n_nodes, n_edges, feat):
    nw = _NC * _NS
    epw = n_edges // nw
    nchunks = epw // _CHUNK
    npad = _node_pad(n_nodes)
    rps = npad // _NS
    niw = rps // _CR

    mesh = plsc.VectorSubcoreMesh(core_axis_name="c", subcore_axis_name="s")

    npairs = nchunks // 2

    def body(h_hbm, src3_hbm, dst3_hbm, zf_hbm, agg_hbm,
             sidx, didx, rows0, rows1, stage, agg_sh, sem0, sem1):
        cid = lax.axis_index("c")
        sid = lax.axis_index("s")
        wid = sid * _NC + cid
        rb = sid * rps

        pltpu.sync_copy(src3_hbm.at[wid], sidx)
        pltpu.sync_copy(dst3_hbm.at[wid], didx)

        pltpu.sync_copy(zf_hbm, stage)
        for j in range(niw):
            pltpu.sync_copy(stage, agg_sh.at[pl.ds(rb + j * _CR, _CR)])
        plsc.subcore_barrier()

        def sl(ref, c):
            return ref.at[pl.ds(c * _CHUNK, _CHUNK)]

        pltpu.async_copy(h_hbm.at[sl(sidx, 0)], rows0, sem0)

        @pl.loop(0, npairs)
        def pair(p):
            c0 = 2 * p
            c1 = c0 + 1
            c2 = jnp.minimum(c0 + 2, nchunks - 1)
            pltpu.async_copy(h_hbm.at[sl(sidx, c1)], rows1, sem1)
            pltpu.make_async_copy(h_hbm.at[sl(sidx, c0)], rows0,
                                  sem0).wait()
            pltpu.sync_copy(rows0, agg_sh.at[sl(didx, c0)], add=True)
            pltpu.async_copy(h_hbm.at[sl(sidx, c2)], rows0, sem0)
            pltpu.make_async_copy(h_hbm.at[sl(sidx, c1)], rows1,
                                  sem1).wait()
            pltpu.sync_copy(rows1, agg_sh.at[sl(didx, c1)], add=True)

        pltpu.make_async_copy(h_hbm.at[sl(sidx, 0)], rows0, sem0).wait()
        if nchunks % 2 == 1:
            pltpu.sync_copy(rows0, agg_sh.at[sl(didx, nchunks - 1)],
                            add=True)

        plsc.subcore_barrier()
        for j in range(niw):
            ro = rb + j * _CR
            pltpu.sync_copy(agg_sh.at[pl.ds(ro, _CR)], stage)
            pltpu.sync_copy(stage, agg_hbm.at[pl.ds(cid * npad + ro, _CR)])

    return pl.kernel(
        body,
        out_type=jax.ShapeDtypeStruct((_NC * npad, feat), jnp.float32),
        mesh=mesh,
        scratch_types=[
            pltpu.VMEM((epw,), jnp.int32),
            pltpu.VMEM((epw,), jnp.int32),
            pltpu.VMEM((_CHUNK, feat), jnp.float32),
            pltpu.VMEM((_CHUNK, feat), jnp.float32),
            pltpu.VMEM((_CR, feat), jnp.float32),
            pltpu.VMEM_SHARED((npad, feat), jnp.float32),
            pltpu.SemaphoreType.DMA,
            pltpu.SemaphoreType.DMA,
        ])


def _make_edge_cnt(n_nodes, n_edges):
    nw = _NC * _NS
    epw = n_edges // nw
    nchunks = epw // _CHUNK
    npad = _node_pad(n_nodes)
    rps = npad // _NS
    niw = rps // _CR

    mesh = plsc.VectorSubcoreMesh(core_axis_name="c", subcore_axis_name="s")

    ck = 2 * _CHUNK
    ncc = epw // ck

    def body(dst3_hbm, zc_hbm, ones_hbm, cnt_hbm,
             didx, ones_v, cstage, cnt_sh):
        cid = lax.axis_index("c")
        sid = lax.axis_index("s")
        wid = sid * _NC + cid
        rb = sid * rps

        pltpu.sync_copy(dst3_hbm.at[wid], didx)
        pltpu.sync_copy(zc_hbm, cstage)
        pltpu.sync_copy(ones_hbm, ones_v)
        for j in range(niw):
            pltpu.sync_copy(cstage, cnt_sh.at[pl.ds(rb + j * _CR, _CR)])
        plsc.subcore_barrier()

        @pl.loop(0, ncc)
        def group(g):
            pltpu.sync_copy(ones_v, cnt_sh.at[didx.at[pl.ds(g * ck, ck)]],
                            add=True)

        plsc.subcore_barrier()
        for j in range(niw):
            ro = rb + j * _CR
            pltpu.sync_copy(cnt_sh.at[pl.ds(ro, _CR)], cstage)
            pltpu.sync_copy(cstage, cnt_hbm.at[pl.ds(cid * npad + ro, _CR)])

    return pl.kernel(
        body,
        out_type=jax.ShapeDtypeStruct((_NC * npad, _CW), jnp.float32),
        mesh=mesh,
        scratch_types=[
            pltpu.VMEM((epw,), jnp.int32),
            pltpu.VMEM((2 * _CHUNK, _CW), jnp.float32),
            pltpu.VMEM((_CR, _CW), jnp.float32),
            pltpu.VMEM_SHARED((npad, _CW), jnp.float32),
        ])


def _embed_tc(x2, emb):
    n = x2.shape[0]
    t, e = emb.shape
    blk = 1000

    def body(x_ref, emb_ref, o_ref):
        x = x_ref[...]
        oh = (x == lax.broadcasted_iota(jnp.int32, (blk, t), 1)).astype(
            jnp.float32)
        o_ref[...] = jnp.dot(oh, emb_ref[...],
                             preferred_element_type=jnp.float32,
                     precision=lax.Precision.HIGHEST)

    return pl.pallas_call(
        body,
        grid=(n // blk,),
        in_specs=[pl.BlockSpec((blk, 1), lambda i: (i, 0)),
                  pl.BlockSpec((t, e), lambda i: (0, 0))],
        out_specs=pl.BlockSpec((blk, e), lambda i: (i, 0)),
        out_shape=jax.ShapeDtypeStruct((n, e), jnp.float32),
    )(x2, emb)


def _conv_tc(agg, cnt, h, wl_t, wr_t, b2d):
    n, f = h.shape
    npad = agg.shape[0] // _NC
    blk = 1000

    def body(a_ref, c_ref, h_ref, wl_ref, wr_ref, b_ref, o_ref):
        asum = a_ref[0] + a_ref[1]
        csum = c_ref[0][:, :1] + c_ref[1][:, :1]
        mean = asum / jnp.maximum(csum, 1.0)
        o = jnp.dot(mean, wl_ref[...], preferred_element_type=jnp.float32,
                     precision=lax.Precision.HIGHEST)
        o += jnp.dot(h_ref[...], wr_ref[...],
                     preferred_element_type=jnp.float32,
                     precision=lax.Precision.HIGHEST)
        o_ref[...] = jnp.maximum(o + b_ref[...], 0.0)

    agg3 = agg.reshape(_NC, npad, f)
    cnt3 = cnt.reshape(_NC, npad, _CW)
    return pl.pallas_call(
        body,
        grid=(n // blk,),
        in_specs=[pl.BlockSpec((_NC, blk, f), lambda i: (0, i, 0)),
                  pl.BlockSpec((_NC, blk, _CW), lambda i: (0, i, 0)),
                  pl.BlockSpec((blk, f), lambda i: (i, 0)),
                  pl.BlockSpec((f, f), lambda i: (0, 0)),
                  pl.BlockSpec((f, f), lambda i: (0, 0)),
                  pl.BlockSpec((1, f), lambda i: (0, 0))],
        out_specs=pl.BlockSpec((blk, f), lambda i: (i, 0)),
        out_shape=jax.ShapeDtypeStruct((n, f), jnp.float32),
    )(agg3, cnt3, h, wl_t, wr_t, b2d)


def _conv_pool_tc(agg, cnt, h, wl_t, wr_t, b2d, batch2, num_graphs):
    n, f = h.shape
    npad = agg.shape[0] // _NC
    blk = 1000

    def body(a_ref, c_ref, h_ref, wl_ref, wr_ref, b_ref, bt_ref,
             gs_ref, gc_ref):
        i = pl.program_id(0)
        asum = a_ref[0] + a_ref[1]
        csum = c_ref[0][:, :1] + c_ref[1][:, :1]
        mean = asum / jnp.maximum(csum, 1.0)
        o = jnp.dot(mean, wl_ref[...], preferred_element_type=jnp.float32,
                     precision=lax.Precision.HIGHEST)
        o += jnp.dot(h_ref[...], wr_ref[...],
                     preferred_element_type=jnp.float32,
                     precision=lax.Precision.HIGHEST)
        h2 = jnp.maximum(o + b_ref[...], 0.0)
        bt = bt_ref[...]
        oh = (bt == lax.broadcasted_iota(jnp.int32, (blk, num_graphs),
                                         1)).astype(jnp.float32)
        ps = lax.dot_general(oh, h2, (((0,), (0,)), ((), ())),
                             preferred_element_type=jnp.float32,
                     precision=lax.Precision.HIGHEST)
        pc = jnp.broadcast_to(jnp.sum(oh, axis=0)[:, None], (num_graphs, f))

        @pl.when(i == 0)
        def _():
            gs_ref[...] = ps
            gc_ref[...] = pc

        @pl.when(i != 0)
        def _():
            gs_ref[...] += ps
            gc_ref[...] += pc

    agg3 = agg.reshape(_NC, npad, f)
    cnt3 = cnt.reshape(_NC, npad, _CW)
    return pl.pallas_call(
        body,
        grid=(n // blk,),
        in_specs=[pl.BlockSpec((_NC, blk, f), lambda i: (0, i, 0)),
                  pl.BlockSpec((_NC, blk, _CW), lambda i: (0, i, 0)),
                  pl.BlockSpec((blk, f), lambda i: (i, 0)),
                  pl.BlockSpec((f, f), lambda i: (0, 0)),
                  pl.BlockSpec((f, f), lambda i: (0, 0)),
                  pl.BlockSpec((1, f), lambda i: (0, 0)),
                  pl.BlockSpec((blk, 1), lambda i: (i, 0))],
        out_specs=[pl.BlockSpec((num_graphs, f), lambda i: (0, 0)),
                   pl.BlockSpec((num_graphs, f), lambda i: (0, 0))],
        out_shape=[jax.ShapeDtypeStruct((num_graphs, f), jnp.float32),
                   jax.ShapeDtypeStruct((num_graphs, f), jnp.float32)],
    )(agg3, cnt3, h, wl_t, wr_t, b2d, batch2)


def _mlp_tc(gs, gc, sol, w1g_t, w1s_t, b1_2d, w2_t, b2_2d, w3_t, b3_2d):
    ng, f = gs.shape
    o = w3_t.shape[1]

    def body(gs_ref, gc_ref, sol_ref, w1g, w1s, b1r, w2, b2r, w3, b3r,
             o_ref):
        g = gs_ref[...] / jnp.maximum(gc_ref[...], 1.0)
        z = jnp.dot(g, w1g[...], preferred_element_type=jnp.float32,
                     precision=lax.Precision.HIGHEST)
        z += jnp.dot(sol_ref[...], w1s[...],
                     preferred_element_type=jnp.float32,
                     precision=lax.Precision.HIGHEST)
        z = jnp.maximum(z + b1r[...], 0.0)
        z = jnp.maximum(
            jnp.dot(z, w2[...], preferred_element_type=jnp.float32,
                     precision=lax.Precision.HIGHEST)
            + b2r[...], 0.0)
        o_ref[...] = (jnp.dot(z, w3[...], preferred_element_type=jnp.float32,
                     precision=lax.Precision.HIGHEST)
                      + b3r[...])

    return pl.pallas_call(
        body,
        out_shape=jax.ShapeDtypeStruct((ng, o), jnp.float32),
    )(gs, gc, sol, w1g_t, w1s_t, b1_2d, w2_t, b2_2d, w3_t, b3_2d)


def kernel(x, edge_index, batch, solution_feature, emb, W1l, W1r, b1,
           W2l, W2r, b2, Wm1, bm1, Wm2, bm2, Wm3, bm3):
    n = x.shape[0]
    n_edges = edge_index.shape[1]
    feat = emb.shape[1]
    num_graphs = solution_feature.shape[0]

    x2 = x.astype(jnp.int32).reshape(n, 1)
    nw = _NC * _NS
    src3 = edge_index[0].astype(jnp.int32).reshape(nw, n_edges // nw)
    dst3 = edge_index[1].astype(jnp.int32).reshape(nw, n_edges // nw)
    batch2 = batch.astype(jnp.int32).reshape(n, 1)

    zf = jnp.zeros((_CR, feat), jnp.float32)
    zc = jnp.zeros((_CR, _CW), jnp.float32)
    ones = jnp.ones((2 * _CHUNK, _CW), jnp.float32)

    edge_agg = _make_edge_agg(n, n_edges, feat)
    cnt = _make_edge_cnt(n, n_edges)(dst3, zc, ones)

    h0 = _embed_tc(x2, emb)
    agg1 = edge_agg(h0, src3, dst3, zf)
    h1 = _conv_tc(agg1, cnt, h0, W1l.T, W1r.T, b1.reshape(1, -1))
    agg2 = edge_agg(h1, src3, dst3, zf)
    gs, gc = _conv_pool_tc(agg2, cnt, h1, W2l.T, W2r.T, b2.reshape(1, -1),
                           batch2, num_graphs)
    out = _mlp_tc(gs, gc, solution_feature,
                  Wm1.T[:feat], Wm1.T[feat:], bm1.reshape(1, -1),
                  Wm2.T, bm2.reshape(1, -1),
                  Wm3.T, bm3.reshape(1, -1))
    return out


# speedup vs baseline: 9.5286x; 1.2084x over previous
import jax
import jax.numpy as jnp
from jax import lax
from jax.experimental import pallas as pl
from jax.experimental.pallas import tpu as pltpu
from jax.experimental.pallas import tpu_sc as plsc

_NC = 2
_NS = 16
_CHUNK = 80
_CW = 128
_CR = 80


def _node_pad(n_nodes):
    return -(-n_nodes // (_NS * _CR)) * (_NS * _CR)


def _make_edge_agg(n_nodes, n_edges, feat):
    nw = _NC * _NS
    epw = n_edges // nw
    nchunks = epw // _CHUNK
    npad = _node_pad(n_nodes)
    rps = npad // _NS
    niw = rps // _CR

    mesh = plsc.VectorSubcoreMesh(core_axis_name="c", subcore_axis_name="s")

    npairs = nchunks // 2

    def body(h_hbm, src3_hbm, dst3_hbm, zf_hbm, agg_hbm,
             sidx, didx, rows0, rows1, agg_sh, sem0, sem1):
        stage = rows0
        cid = lax.axis_index("c")
        sid = lax.axis_index("s")
        wid = sid * _NC + cid
        rb = sid * rps

        pltpu.sync_copy(src3_hbm.at[wid], sidx)
        pltpu.sync_copy(dst3_hbm.at[wid], didx)

        pltpu.sync_copy(zf_hbm, stage)
        for j in range(niw):
            pltpu.sync_copy(stage, agg_sh.at[pl.ds(rb + j * _CR, _CR)])
        plsc.subcore_barrier()

        def sl(ref, c):
            return ref.at[pl.ds(c * _CHUNK, _CHUNK)]

        pltpu.async_copy(h_hbm.at[sl(sidx, 0)], rows0, sem0)

        @pl.loop(0, npairs)
        def pair(p):
            c0 = 2 * p
            c1 = c0 + 1
            c2 = jnp.minimum(c0 + 2, nchunks - 1)
            pltpu.async_copy(h_hbm.at[sl(sidx, c1)], rows1, sem1)
            pltpu.make_async_copy(h_hbm.at[sl(sidx, c0)], rows0,
                                  sem0).wait()
            pltpu.sync_copy(rows0, agg_sh.at[sl(didx, c0)], add=True)
            pltpu.async_copy(h_hbm.at[sl(sidx, c2)], rows0, sem0)
            pltpu.make_async_copy(h_hbm.at[sl(sidx, c1)], rows1,
                                  sem1).wait()
            pltpu.sync_copy(rows1, agg_sh.at[sl(didx, c1)], add=True)

        pltpu.make_async_copy(h_hbm.at[sl(sidx, 0)], rows0, sem0).wait()
        if nchunks % 2 == 1:
            pltpu.sync_copy(rows0, agg_sh.at[sl(didx, nchunks - 1)],
                            add=True)

        plsc.subcore_barrier()
        for j in range(niw):
            ro = rb + j * _CR
            pltpu.sync_copy(agg_sh.at[pl.ds(ro, _CR)], stage)
            pltpu.sync_copy(stage, agg_hbm.at[pl.ds(cid * npad + ro, _CR)])

    return pl.kernel(
        body,
        out_type=jax.ShapeDtypeStruct((_NC * npad, feat), jnp.float32),
        mesh=mesh,
        scratch_types=[
            pltpu.VMEM((epw,), jnp.int32),
            pltpu.VMEM((epw,), jnp.int32),
            pltpu.VMEM((_CHUNK, feat), jnp.float32),
            pltpu.VMEM((_CHUNK, feat), jnp.float32),
            pltpu.VMEM_SHARED((npad, feat), jnp.float32),
            pltpu.SemaphoreType.DMA,
            pltpu.SemaphoreType.DMA,
        ])


def _make_edge_cnt(n_nodes, n_edges):
    nw = _NC * _NS
    epw = n_edges // nw
    nchunks = epw // _CHUNK
    npad = _node_pad(n_nodes)
    rps = npad // _NS
    niw = rps // _CR

    mesh = plsc.VectorSubcoreMesh(core_axis_name="c", subcore_axis_name="s")

    ck = _CHUNK
    ncc = epw // ck

    def body(dst3_hbm, zc_hbm, ones_hbm, cnt_hbm,
             didx, ones_v, cstage, cnt_sh):
        cid = lax.axis_index("c")
        sid = lax.axis_index("s")
        wid = sid * _NC + cid
        rb = sid * rps

        pltpu.sync_copy(dst3_hbm.at[wid], didx)
        pltpu.sync_copy(zc_hbm, cstage)
        pltpu.sync_copy(ones_hbm, ones_v)
        for j in range(niw):
            pltpu.sync_copy(cstage, cnt_sh.at[pl.ds(rb + j * _CR, _CR)])
        plsc.subcore_barrier()

        @pl.loop(0, ncc)
        def group(g):
            pltpu.sync_copy(ones_v, cnt_sh.at[didx.at[pl.ds(g * ck, ck)]],
                            add=True)

        plsc.subcore_barrier()
        for j in range(niw):
            ro = rb + j * _CR
            pltpu.sync_copy(cnt_sh.at[pl.ds(ro, _CR)], cstage)
            pltpu.sync_copy(cstage, cnt_hbm.at[pl.ds(cid * npad + ro, _CR)])

    return pl.kernel(
        body,
        out_type=jax.ShapeDtypeStruct((_NC * npad, _CW), jnp.float32),
        mesh=mesh,
        scratch_types=[
            pltpu.VMEM((epw,), jnp.int32),
            pltpu.VMEM((_CHUNK, _CW), jnp.float32),
            pltpu.VMEM((_CR, _CW), jnp.float32),
            pltpu.VMEM_SHARED((npad, _CW), jnp.float32),
        ])


def _embed_tc(x2, emb):
    n = x2.shape[0]
    t, e = emb.shape
    blk = 1000

    def body(x_ref, emb_ref, o_ref):
        x = x_ref[...]
        oh = (x == lax.broadcasted_iota(jnp.int32, (blk, t), 1)).astype(
            jnp.float32)
        o_ref[...] = jnp.dot(oh, emb_ref[...],
                             preferred_element_type=jnp.float32,
                     precision=lax.Precision.HIGHEST)

    return pl.pallas_call(
        body,
        grid=(n // blk,),
        in_specs=[pl.BlockSpec((blk, 1), lambda i: (i, 0)),
                  pl.BlockSpec((t, e), lambda i: (0, 0))],
        out_specs=pl.BlockSpec((blk, e), lambda i: (i, 0)),
        out_shape=jax.ShapeDtypeStruct((n, e), jnp.float32),
    )(x2, emb)


def _conv_tc(agg, cnt, h, wl_t, wr_t, b2d):
    n, f = h.shape
    npad = agg.shape[0] // _NC
    blk = 1000

    def body(a_ref, c_ref, h_ref, wl_ref, wr_ref, b_ref, o_ref):
        asum = a_ref[0] + a_ref[1]
        csum = c_ref[0][:, :1] + c_ref[1][:, :1]
        mean = asum / jnp.maximum(csum, 1.0)
        o = jnp.dot(mean, wl_ref[...], preferred_element_type=jnp.float32,
                     precision=lax.Precision.HIGHEST)
        o += jnp.dot(h_ref[...], wr_ref[...],
                     preferred_element_type=jnp.float32,
                     precision=lax.Precision.HIGHEST)
        o_ref[...] = jnp.maximum(o + b_ref[...], 0.0)

    agg3 = agg.reshape(_NC, npad, f)
    cnt3 = cnt.reshape(_NC, npad, _CW)
    return pl.pallas_call(
        body,
        grid=(n // blk,),
        in_specs=[pl.BlockSpec((_NC, blk, f), lambda i: (0, i, 0)),
                  pl.BlockSpec((_NC, blk, _CW), lambda i: (0, i, 0)),
                  pl.BlockSpec((blk, f), lambda i: (i, 0)),
                  pl.BlockSpec((f, f), lambda i: (0, 0)),
                  pl.BlockSpec((f, f), lambda i: (0, 0)),
                  pl.BlockSpec((1, f), lambda i: (0, 0))],
        out_specs=pl.BlockSpec((blk, f), lambda i: (i, 0)),
        out_shape=jax.ShapeDtypeStruct((n, f), jnp.float32),
    )(agg3, cnt3, h, wl_t, wr_t, b2d)


def _conv_pool_tc(agg, cnt, h, wl_t, wr_t, b2d, batch2, num_graphs):
    n, f = h.shape
    npad = agg.shape[0] // _NC
    blk = 1000

    def body(a_ref, c_ref, h_ref, wl_ref, wr_ref, b_ref, bt_ref,
             gs_ref, gc_ref):
        i = pl.program_id(0)
        asum = a_ref[0] + a_ref[1]
        csum = c_ref[0][:, :1] + c_ref[1][:, :1]
        mean = asum / jnp.maximum(csum, 1.0)
        o = jnp.dot(mean, wl_ref[...], preferred_element_type=jnp.float32,
                     precision=lax.Precision.HIGHEST)
        o += jnp.dot(h_ref[...], wr_ref[...],
                     preferred_element_type=jnp.float32,
                     precision=lax.Precision.HIGHEST)
        h2 = jnp.maximum(o + b_ref[...], 0.0)
        bt = bt_ref[...]
        oh = (bt == lax.broadcasted_iota(jnp.int32, (blk, num_graphs),
                                         1)).astype(jnp.float32)
        ps = lax.dot_general(oh, h2, (((0,), (0,)), ((), ())),
                             preferred_element_type=jnp.float32,
                     precision=lax.Precision.HIGHEST)
        pc = jnp.broadcast_to(jnp.sum(oh, axis=0)[:, None], (num_graphs, f))

        @pl.when(i == 0)
        def _():
            gs_ref[...] = ps
            gc_ref[...] = pc

        @pl.when(i != 0)
        def _():
            gs_ref[...] += ps
            gc_ref[...] += pc

    agg3 = agg.reshape(_NC, npad, f)
    cnt3 = cnt.reshape(_NC, npad, _CW)
    return pl.pallas_call(
        body,
        grid=(n // blk,),
        in_specs=[pl.BlockSpec((_NC, blk, f), lambda i: (0, i, 0)),
                  pl.BlockSpec((_NC, blk, _CW), lambda i: (0, i, 0)),
                  pl.BlockSpec((blk, f), lambda i: (i, 0)),
                  pl.BlockSpec((f, f), lambda i: (0, 0)),
                  pl.BlockSpec((f, f), lambda i: (0, 0)),
                  pl.BlockSpec((1, f), lambda i: (0, 0)),
                  pl.BlockSpec((blk, 1), lambda i: (i, 0))],
        out_specs=[pl.BlockSpec((num_graphs, f), lambda i: (0, 0)),
                   pl.BlockSpec((num_graphs, f), lambda i: (0, 0))],
        out_shape=[jax.ShapeDtypeStruct((num_graphs, f), jnp.float32),
                   jax.ShapeDtypeStruct((num_graphs, f), jnp.float32)],
    )(agg3, cnt3, h, wl_t, wr_t, b2d, batch2)


def _mlp_tc(gs, gc, sol, w1g_t, w1s_t, b1_2d, w2_t, b2_2d, w3_t, b3_2d):
    ng, f = gs.shape
    o = w3_t.shape[1]

    def body(gs_ref, gc_ref, sol_ref, w1g, w1s, b1r, w2, b2r, w3, b3r,
             o_ref):
        g = gs_ref[...] / jnp.maximum(gc_ref[...], 1.0)
        z = jnp.dot(g, w1g[...], preferred_element_type=jnp.float32,
                     precision=lax.Precision.HIGHEST)
        z += jnp.dot(sol_ref[...], w1s[...],
                     preferred_element_type=jnp.float32,
                     precision=lax.Precision.HIGHEST)
        z = jnp.maximum(z + b1r[...], 0.0)
        z = jnp.maximum(
            jnp.dot(z, w2[...], preferred_element_type=jnp.float32,
                     precision=lax.Precision.HIGHEST)
            + b2r[...], 0.0)
        o_ref[...] = (jnp.dot(z, w3[...], preferred_element_type=jnp.float32,
                     precision=lax.Precision.HIGHEST)
                      + b3r[...])

    return pl.pallas_call(
        body,
        out_shape=jax.ShapeDtypeStruct((ng, o), jnp.float32),
    )(gs, gc, sol, w1g_t, w1s_t, b1_2d, w2_t, b2_2d, w3_t, b3_2d)


def kernel(x, edge_index, batch, solution_feature, emb, W1l, W1r, b1,
           W2l, W2r, b2, Wm1, bm1, Wm2, bm2, Wm3, bm3):
    n = x.shape[0]
    n_edges = edge_index.shape[1]
    feat = emb.shape[1]
    num_graphs = solution_feature.shape[0]

    x2 = x.astype(jnp.int32).reshape(n, 1)
    nw = _NC * _NS
    src3 = edge_index[0].astype(jnp.int32).reshape(nw, n_edges // nw)
    dst3 = edge_index[1].astype(jnp.int32).reshape(nw, n_edges // nw)
    batch2 = batch.astype(jnp.int32).reshape(n, 1)

    zf = jnp.zeros((_CR, feat), jnp.float32)
    zc = jnp.zeros((_CR, _CW), jnp.float32)
    ones = jnp.ones((_CHUNK, _CW), jnp.float32)

    edge_agg = _make_edge_agg(n, n_edges, feat)
    cnt = _make_edge_cnt(n, n_edges)(dst3, zc, ones)

    h0 = _embed_tc(x2, emb)
    agg1 = edge_agg(h0, src3, dst3, zf)
    h1 = _conv_tc(agg1, cnt, h0, W1l.T, W1r.T, b1.reshape(1, -1))
    agg2 = edge_agg(h1, src3, dst3, zf)
    gs, gc = _conv_pool_tc(agg2, cnt, h1, W2l.T, W2r.T, b2.reshape(1, -1),
                           batch2, num_graphs)
    out = _mlp_tc(gs, gc, solution_feature,
                  Wm1.T[:feat], Wm1.T[feat:], bm1.reshape(1, -1),
                  Wm2.T, bm2.reshape(1, -1),
                  Wm3.T, bm3.reshape(1, -1))
    return out
